# baseline (speedup 1.0000x reference)
import functools

import numpy as np
import jax
import jax.numpy as jnp
from jax import lax
from jax.experimental import pallas as pl
from jax.experimental.pallas import tpu as pltpu
from jax.experimental.pallas import tpu_sc as plsc

NC, NS, LANES = 2, 16, 16
NW = NC * NS

B, L, D = 16384, 200, 128
SPW = B // NW
IDX_CHUNK = 100
NCH = L // IDX_CHUNK
BLK = 32
NBLK = SPW // BLK
NSLOT = 4

_PERM = np.array(
    [32 * (q // 2) + 2 * l + (q % 2) for q in range(8) for l in range(16)]
)


def _bag_body(idx_hbm, tab_hbm, x_hbm, idx_v, acc_v, *bufs_and_sems):
    bufs = bufs_and_sems[: NSLOT * NCH]
    sems = bufs_and_sems[NSLOT * NCH:]
    cid = lax.axis_index("c")
    sid = lax.axis_index("s")
    wid = sid * NC + cid

    def fire(t, slot):
        bank = lax.rem(lax.div(t, BLK), 2)
        row0 = lax.rem(t, BLK) * NCH
        for c in range(NCH):
            pltpu.async_copy(tab_hbm.at[idx_v.at[bank].at[row0 + c]],
                             bufs[slot * NCH + c], sems[slot])

    def drain(slot):
        for c in range(NCH):
            pltpu.make_async_copy(tab_hbm.at[idx_v.at[0].at[0]],
                                  bufs[slot * NCH + c], sems[slot]).wait()

    def stage_idx(nb):
        bank = lax.rem(nb, 2)
        pltpu.sync_copy(idx_hbm.at[wid * NBLK + nb], idx_v.at[bank])

    hi_mask = jnp.full((LANES,), -65536, jnp.int32)
    shift16 = jnp.full((LANES,), 16, jnp.int32)

    def accum(s, slot):
        def r_body(r, accs):
            out = list(accs)
            for c in range(NCH):
                for k in range(D // 32):
                    v = bufs[slot * NCH + c][r, pl.ds(k * LANES, LANES)]
                    ev = plsc.bitcast(lax.shift_left(v, shift16), jnp.float32)
                    od = plsc.bitcast(jnp.bitwise_and(v, hi_mask), jnp.float32)
                    out[2 * k] = out[2 * k] + ev
                    out[2 * k + 1] = out[2 * k + 1] + od
            return tuple(out)

        init = tuple(jnp.zeros((LANES,), jnp.float32) for _ in range(D // LANES))
        accs = lax.fori_loop(0, IDX_CHUNK, r_body, init)
        arow = lax.rem(s, BLK)
        for k in range(D // LANES):
            acc_v[arow, pl.ds(k * LANES, LANES)] = accs[k]

    stage_idx(0)
    for b in range(NSLOT):
        fire(b, b)

    def g_body(g2, carry):
        g = g2 * NSLOT
        for b in range(NSLOT):
            s = g + b
            t = s + NSLOT
            drain(b)
            accum(s, b)

            @pl.when(jnp.logical_and(lax.rem(t, BLK) == 0, t < SPW))
            def _():
                stage_idx(t // BLK)

            @pl.when(t < SPW)
            def _():
                fire(t, b)

        s_last = g + NSLOT - 1

        @pl.when(lax.rem(s_last, BLK) == BLK - 1)
        def _():
            base = pl.multiple_of(wid * SPW + s_last - (BLK - 1), BLK)
            pltpu.sync_copy(acc_v, x_hbm.at[pl.ds(base, BLK)])

        return carry

    lax.fori_loop(0, SPW // NSLOT, g_body, 0)


def _embedding_bag(indices, emb_table):
    tab_i32 = lax.bitcast_convert_type(
        emb_table.astype(jnp.bfloat16).reshape(-1, D // 2, 2), jnp.int32)
    idx3 = indices.astype(jnp.int32).reshape(B // BLK, BLK * NCH, IDX_CHUNK)
    mesh = plsc.VectorSubcoreMesh(
        core_axis_name="c", subcore_axis_name="s", num_cores=NC, num_subcores=NS
    )
    return pl.kernel(
        _bag_body,
        out_type=jax.ShapeDtypeStruct((B, D), jnp.float32),
        mesh=mesh,
        compiler_params=pltpu.CompilerParams(
            needs_layout_passes=False, use_tc_tiling_on_sc=False),
        scratch_types=[
            pltpu.VMEM((2, BLK * NCH, IDX_CHUNK), jnp.int32),
            pltpu.VMEM((BLK, D), jnp.float32),
        ] + [pltpu.VMEM((IDX_CHUNK, D // 2), jnp.int32)] * (NSLOT * NCH)
          + [pltpu.SemaphoreType.DMA] * NSLOT,
    )(idx3, tab_i32)


TB = 1024


def _head_body(x_ref, pwt_ref, pb_ref, w1_ref, b1_ref, w2_ref, b2_ref,
               w3_ref, b3_ref, pol_ref, val_ref):
    x = x_ref[...]
    hi = lax.Precision.HIGHEST
    pol_ref[...] = (
        lax.dot_general(x, pwt_ref[...], (((1,), (0,)), ((), ())), precision=hi)
        + pb_ref[...]
    )
    h = jnp.clip(
        lax.dot_general(x, w1_ref[...], (((1,), (0,)), ((), ())), precision=hi)
        + b1_ref[...], 0.0, 1.0)
    h = jnp.clip(
        lax.dot_general(h, w2_ref[...], (((1,), (0,)), ((), ())), precision=hi)
        + b2_ref[...], 0.0, 1.0)
    val_ref[...] = jnp.tanh(
        lax.dot_general(h, w3_ref[...], (((1,), (0,)), ((), ())), precision=hi)
        + b3_ref[...])


def _heads(x, pw, pb, v1w, v1b, v2w, v2b, v3w, v3b):
    np_ = pw.shape[0]
    full = lambda shape: pl.BlockSpec(shape, lambda i: (0, 0))
    return pl.pallas_call(
        _head_body,
        grid=(B // TB,),
        in_specs=[
            pl.BlockSpec((TB, D), lambda i: (i, 0)),
            full((D, np_)),
            full((1, np_)),
            full((D, 32)),
            full((1, 32)),
            full((32, 32)),
            full((1, 32)),
            full((32, 1)),
            full((1, 1)),
        ],
        out_specs=[
            pl.BlockSpec((TB, np_), lambda i: (i, 0)),
            pl.BlockSpec((TB, 1), lambda i: (i, 0)),
        ],
        out_shape=[
            jax.ShapeDtypeStruct((B, np_), jnp.float32),
            jax.ShapeDtypeStruct((B, 1), jnp.float32),
        ],
    )(
        x, pw.T[_PERM], pb.reshape(1, np_), v1w.T[_PERM], v1b.reshape(1, 32),
        v2w.T, v2b.reshape(1, 32), v3w.T, v3b.reshape(1, 1),
    )


def kernel(indices, emb_table, pw, pb, v1w, v1b, v2w, v2b, v3w, v3b):
    x = _embedding_bag(indices, emb_table)
    pol, val = _heads(x, pw, pb, v1w, v1b, v2w, v2b, v3w, v3b)
    return (pol, val)


# speedup vs baseline: 1.5651x; 1.5651x over previous
import functools

import numpy as np
import jax
import jax.numpy as jnp
from jax import lax
from jax.experimental import pallas as pl
from jax.experimental.pallas import tpu as pltpu
from jax.experimental.pallas import tpu_sc as plsc

NC, NS, LANES = 2, 16, 16
NW = NC * NS

B, L, D = 16384, 200, 128
F = 2 * 225 * 225
SPW = B // NW
IDX_CHUNK = 100
NCH = L // IDX_CHUNK
BLK = 32
NBLK = SPW // BLK
NSLOT = 4

RPB = 64
NFULL = F // RPB
REM = F - NFULL * RPB
NITER_A = (NFULL + NW - 1) // NW

_SC_PARAMS = pltpu.CompilerParams(
    needs_layout_passes=False, use_tc_tiling_on_sc=False)


def _pack_body(tab_hbm, out_hbm, in_v, out_v):
    cid = lax.axis_index("c")
    sid = lax.axis_index("s")
    wid = sid * NC + cid
    hi_mask = jnp.full((LANES,), -65536, jnp.int32)
    s16 = jnp.full((LANES,), 16, jnp.int32)
    one = jnp.full((LANES,), 1, jnp.int32)
    half = jnp.full((LANES,), 0x7FFF, jnp.int32)

    def rne(bits):
        return bits + half + jnp.bitwise_and(
            lax.shift_right_logical(bits, s16), one)

    def pack_rows(nrows):
        def r_body(r, carry):
            for k in range(D // 32):
                a = in_v[r, pl.ds(k * 32, LANES)]
                bb = in_v[r, pl.ds(k * 32 + 16, LANES)]
                ra = rne(plsc.bitcast(a, jnp.int32))
                rb = rne(plsc.bitcast(bb, jnp.int32))
                out_v[r, pl.ds(k * LANES, LANES)] = jnp.bitwise_or(
                    lax.shift_right_logical(ra, s16),
                    jnp.bitwise_and(rb, hi_mask))
            return carry

        lax.fori_loop(0, nrows, r_body, 0)

    def c_body(i, carry):
        t = wid + i * NW

        @pl.when(t < NFULL)
        def _():
            pltpu.sync_copy(tab_hbm.at[pl.ds(t * RPB, RPB)], in_v)
            pack_rows(RPB)
            pltpu.sync_copy(out_v, out_hbm.at[pl.ds(t * RPB, RPB)])

        return carry

    lax.fori_loop(0, NITER_A, c_body, 0)

    @pl.when(wid == 0)
    def _():
        pltpu.sync_copy(tab_hbm.at[pl.ds(NFULL * RPB, REM)],
                        in_v.at[pl.ds(0, REM)])
        pack_rows(REM)
        pltpu.sync_copy(out_v.at[pl.ds(0, REM)],
                        out_hbm.at[pl.ds(NFULL * RPB, REM)])


def _bag_body(idx_hbm, tab_hbm, x_hbm, idx_v, acc_v, *bufs_and_sems):
    bufs = bufs_and_sems[: NSLOT * NCH]
    sems = bufs_and_sems[NSLOT * NCH:]
    cid = lax.axis_index("c")
    sid = lax.axis_index("s")
    wid = sid * NC + cid

    def fire(t, slot):
        bank = lax.rem(lax.div(t, BLK), 2)
        row0 = lax.rem(t, BLK) * NCH
        for c in range(NCH):
            pltpu.async_copy(tab_hbm.at[idx_v.at[bank].at[row0 + c]],
                             bufs[slot * NCH + c], sems[slot])

    def drain(slot):
        for c in range(NCH):
            pltpu.make_async_copy(tab_hbm.at[idx_v.at[0].at[0]],
                                  bufs[slot * NCH + c], sems[slot]).wait()

    def stage_idx(nb):
        bank = lax.rem(nb, 2)
        pltpu.sync_copy(idx_hbm.at[wid * NBLK + nb], idx_v.at[bank])

    hi_mask = jnp.full((LANES,), -65536, jnp.int32)
    shift16 = jnp.full((LANES,), 16, jnp.int32)

    def accum(s, slot):
        def r_body(r, accs):
            out = list(accs)
            for c in range(NCH):
                for k in range(D // 32):
                    v = bufs[slot * NCH + c][r, pl.ds(k * LANES, LANES)]
                    ev = plsc.bitcast(lax.shift_left(v, shift16), jnp.float32)
                    od = plsc.bitcast(jnp.bitwise_and(v, hi_mask), jnp.float32)
                    out[2 * k] = out[2 * k] + ev
                    out[2 * k + 1] = out[2 * k + 1] + od
            return tuple(out)

        init = tuple(jnp.zeros((LANES,), jnp.float32) for _ in range(D // LANES))
        accs = lax.fori_loop(0, IDX_CHUNK, r_body, init)
        arow = lax.rem(s, BLK)
        for k in range(D // LANES):
            acc_v[arow, pl.ds(k * LANES, LANES)] = accs[k]

    stage_idx(0)
    for b in range(NSLOT):
        fire(b, b)

    def g_body(g2, carry):
        g = g2 * NSLOT
        for b in range(NSLOT):
            s = g + b
            t = s + NSLOT
            drain(b)
            accum(s, b)

            @pl.when(jnp.logical_and(lax.rem(t, BLK) == 0, t < SPW))
            def _():
                stage_idx(t // BLK)

            @pl.when(t < SPW)
            def _():
                fire(t, b)

        s_last = g + NSLOT - 1

        @pl.when(lax.rem(s_last, BLK) == BLK - 1)
        def _():
            base = pl.multiple_of(wid * SPW + s_last - (BLK - 1), BLK)
            pltpu.sync_copy(acc_v, x_hbm.at[pl.ds(base, BLK)])

        return carry

    lax.fori_loop(0, SPW // NSLOT, g_body, 0)


def _embedding_bag(indices, emb_table):
    idx3 = indices.astype(jnp.int32).reshape(B // BLK, BLK * NCH, IDX_CHUNK)
    mesh = plsc.VectorSubcoreMesh(
        core_axis_name="c", subcore_axis_name="s", num_cores=NC, num_subcores=NS
    )
    tabp = pl.kernel(
        _pack_body,
        out_type=jax.ShapeDtypeStruct((F, D // 2), jnp.int32),
        mesh=mesh,
        compiler_params=_SC_PARAMS,
        scratch_types=[
            pltpu.VMEM((RPB, D), jnp.float32),
            pltpu.VMEM((RPB, D // 2), jnp.int32),
        ],
    )(emb_table)
    return pl.kernel(
        _bag_body,
        out_type=jax.ShapeDtypeStruct((B, D), jnp.float32),
        mesh=mesh,
        compiler_params=_SC_PARAMS,
        scratch_types=[
            pltpu.VMEM((2, BLK * NCH, IDX_CHUNK), jnp.int32),
            pltpu.VMEM((BLK, D), jnp.float32),
        ] + [pltpu.VMEM((IDX_CHUNK, D // 2), jnp.int32)] * (NSLOT * NCH)
          + [pltpu.SemaphoreType.DMA] * NSLOT,
    )(idx3, tabp)


TB = 1024


def _head_body(x_ref, pwt_ref, pb_ref, w1_ref, b1_ref, w2_ref, b2_ref,
               w3_ref, b3_ref, pol_ref, val_ref):
    x = x_ref[...]
    hi = lax.Precision.HIGHEST
    pol_ref[...] = (
        lax.dot_general(x, pwt_ref[...], (((1,), (0,)), ((), ())), precision=hi)
        + pb_ref[...]
    )
    h = jnp.clip(
        lax.dot_general(x, w1_ref[...], (((1,), (0,)), ((), ())), precision=hi)
        + b1_ref[...], 0.0, 1.0)
    h = jnp.clip(
        lax.dot_general(h, w2_ref[...], (((1,), (0,)), ((), ())), precision=hi)
        + b2_ref[...], 0.0, 1.0)
    val_ref[...] = jnp.tanh(
        lax.dot_general(h, w3_ref[...], (((1,), (0,)), ((), ())), precision=hi)
        + b3_ref[...])


def _heads(x, pw, pb, v1w, v1b, v2w, v2b, v3w, v3b):
    np_ = pw.shape[0]
    full = lambda shape: pl.BlockSpec(shape, lambda i: (0, 0))
    return pl.pallas_call(
        _head_body,
        grid=(B // TB,),
        in_specs=[
            pl.BlockSpec((TB, D), lambda i: (i, 0)),
            full((D, np_)),
            full((1, np_)),
            full((D, 32)),
            full((1, 32)),
            full((32, 32)),
            full((1, 32)),
            full((32, 1)),
            full((1, 1)),
        ],
        out_specs=[
            pl.BlockSpec((TB, np_), lambda i: (i, 0)),
            pl.BlockSpec((TB, 1), lambda i: (i, 0)),
        ],
        out_shape=[
            jax.ShapeDtypeStruct((B, np_), jnp.float32),
            jax.ShapeDtypeStruct((B, 1), jnp.float32),
        ],
    )(
        x, pw.T, pb.reshape(1, np_), v1w.T, v1b.reshape(1, 32),
        v2w.T, v2b.reshape(1, 32), v3w.T, v3b.reshape(1, 1),
    )


def kernel(indices, emb_table, pw, pb, v1w, v1b, v2w, v2b, v3w, v3b):
    x = _embedding_bag(indices, emb_table)
    pol, val = _heads(x, pw, pb, v1w, v1b, v2w, v2b, v3w, v3b)
    return (pol, val)


# speedup vs baseline: 1.7709x; 1.1315x over previous
import functools

import numpy as np
import jax
import jax.numpy as jnp
from jax import lax
from jax.experimental import pallas as pl
from jax.experimental.pallas import tpu as pltpu
from jax.experimental.pallas import tpu_sc as plsc

NC, NS, LANES = 2, 16, 16
NW = NC * NS

B, L, D = 16384, 200, 128
F = 2 * 225 * 225
SPW = B // NW
IDX_CHUNK = 100
NCH = L // IDX_CHUNK
BLK = 32
NBLK = SPW // BLK
NSLOT = 4

RPB = 128
NFULL = F // RPB
REM = F - NFULL * RPB
NITER_A = (NFULL + NW - 1) // NW

_SC_PARAMS = pltpu.CompilerParams(
    needs_layout_passes=False, use_tc_tiling_on_sc=False)


def _pack_body(tab_hbm, out_hbm, in_v, out_v, sem_i0, sem_i1, sem_o0, sem_o1):
    cid = lax.axis_index("c")
    sid = lax.axis_index("s")
    wid = sid * NC + cid
    sem_i = (sem_i0, sem_i1)
    sem_o = (sem_o0, sem_o1)
    hi_mask = jnp.full((LANES,), -65536, jnp.int32)
    s16 = jnp.full((LANES,), 16, jnp.int32)
    one = jnp.full((LANES,), 1, jnp.int32)
    half = jnp.full((LANES,), 0x7FFF, jnp.int32)

    def rne(bits):
        return bits + half + jnp.bitwise_and(
            lax.shift_right_logical(bits, s16), one)

    def pack_rows(slot, nrows):
        def r_body(r, carry):
            for k in range(D // 32):
                a = in_v[slot, r, pl.ds(k * 32, LANES)]
                bb = in_v[slot, r, pl.ds(k * 32 + 16, LANES)]
                ra = rne(plsc.bitcast(a, jnp.int32))
                rb = rne(plsc.bitcast(bb, jnp.int32))
                out_v[slot, r, pl.ds(k * LANES, LANES)] = jnp.bitwise_or(
                    lax.shift_right_logical(ra, s16),
                    jnp.bitwise_and(rb, hi_mask))
            return carry

        lax.fori_loop(0, nrows, r_body, 0)

    def fire_in(t, slot):
        pltpu.async_copy(tab_hbm.at[pl.ds(t * RPB, RPB)],
                         in_v.at[slot], sem_i[slot])

    def c_body(i, slot):
        t = wid + i * NW

        @pl.when(t + NW < NFULL)
        def _():
            fire_in(t + NW, 1 - slot)

        @pl.when(t < NFULL)
        def _():
            pltpu.make_async_copy(tab_hbm.at[pl.ds(0, RPB)],
                                  in_v.at[slot], sem_i[slot]).wait()

            @pl.when(t - 2 * NW >= 0)
            def _():
                pltpu.make_async_copy(out_v.at[slot],
                                      out_hbm.at[pl.ds(0, RPB)],
                                      sem_o[slot]).wait()

            pack_rows(slot, RPB)
            pltpu.async_copy(out_v.at[slot],
                             out_hbm.at[pl.ds(t * RPB, RPB)], sem_o[slot])

    fire_in(wid, 0)
    def c2_body(i2, carry):
        c_body(2 * i2, 0)
        c_body(2 * i2 + 1, 1)
        return carry

    assert NITER_A % 2 == 1
    lax.fori_loop(0, NITER_A // 2, c2_body, 0)
    c_body(NITER_A - 1, (NITER_A - 1) % 2)

    for j in (NITER_A - 2, NITER_A - 1):
        t = wid + j * NW

        @pl.when(t < NFULL)
        def _():
            pltpu.make_async_copy(out_v.at[j % 2],
                                  out_hbm.at[pl.ds(0, RPB)],
                                  sem_o[j % 2]).wait()

    @pl.when(wid == 0)
    def _():
        pltpu.sync_copy(tab_hbm.at[pl.ds(NFULL * RPB, REM)],
                        in_v.at[0].at[pl.ds(0, REM)])
        pack_rows(0, REM)
        pltpu.sync_copy(out_v.at[0].at[pl.ds(0, REM)],
                        out_hbm.at[pl.ds(NFULL * RPB, REM)])


def _bag_body(idx_hbm, tab_hbm, x_hbm, idx_v, acc_v, *bufs_and_sems):
    bufs = bufs_and_sems[: NSLOT * NCH]
    sems = bufs_and_sems[NSLOT * NCH:]
    cid = lax.axis_index("c")
    sid = lax.axis_index("s")
    wid = sid * NC + cid

    def fire(t, slot):
        bank = lax.rem(lax.div(t, BLK), 2)
        row0 = lax.rem(t, BLK) * NCH
        for c in range(NCH):
            pltpu.async_copy(tab_hbm.at[idx_v.at[bank].at[row0 + c]],
                             bufs[slot * NCH + c], sems[slot])

    def drain(slot):
        for c in range(NCH):
            pltpu.make_async_copy(tab_hbm.at[idx_v.at[0].at[0]],
                                  bufs[slot * NCH + c], sems[slot]).wait()

    def stage_idx(nb):
        bank = lax.rem(nb, 2)
        pltpu.sync_copy(idx_hbm.at[wid * NBLK + nb], idx_v.at[bank])

    hi_mask = jnp.full((LANES,), -65536, jnp.int32)
    shift16 = jnp.full((LANES,), 16, jnp.int32)

    def accum(s, slot):
        def r_body(r2, accs):
            out = list(accs)
            for rr in range(2):
                for c in range(NCH):
                    for k in range(D // 32):
                        v = bufs[slot * NCH + c][2 * r2 + rr,
                                                 pl.ds(k * LANES, LANES)]
                        ev = plsc.bitcast(lax.shift_left(v, shift16), jnp.float32)
                        od = plsc.bitcast(jnp.bitwise_and(v, hi_mask), jnp.float32)
                        out[2 * k] = out[2 * k] + ev
                        out[2 * k + 1] = out[2 * k + 1] + od
            return tuple(out)

        init = tuple(jnp.zeros((LANES,), jnp.float32) for _ in range(D // LANES))
        accs = lax.fori_loop(0, IDX_CHUNK // 2, r_body, init)
        arow = lax.rem(s, BLK)
        for k in range(D // LANES):
            acc_v[arow, pl.ds(k * LANES, LANES)] = accs[k]

    stage_idx(0)
    for b in range(NSLOT):
        fire(b, b)

    def g_body(g2, carry):
        g = g2 * NSLOT
        for b in range(NSLOT):
            s = g + b
            t = s + NSLOT
            drain(b)
            accum(s, b)

            @pl.when(jnp.logical_and(lax.rem(t, BLK) == 0, t < SPW))
            def _():
                stage_idx(t // BLK)

            @pl.when(t < SPW)
            def _():
                fire(t, b)

        s_last = g + NSLOT - 1

        @pl.when(lax.rem(s_last, BLK) == BLK - 1)
        def _():
            base = pl.multiple_of(wid * SPW + s_last - (BLK - 1), BLK)
            pltpu.sync_copy(acc_v, x_hbm.at[pl.ds(base, BLK)])

        return carry

    lax.fori_loop(0, SPW // NSLOT, g_body, 0)


def _embedding_bag(indices, emb_table):
    idx3 = indices.astype(jnp.int32).reshape(B // BLK, BLK * NCH, IDX_CHUNK)
    mesh = plsc.VectorSubcoreMesh(
        core_axis_name="c", subcore_axis_name="s", num_cores=NC, num_subcores=NS
    )
    tabp = pl.kernel(
        _pack_body,
        out_type=jax.ShapeDtypeStruct((F, D // 2), jnp.int32),
        mesh=mesh,
        compiler_params=_SC_PARAMS,
        scratch_types=[
            pltpu.VMEM((2, RPB, D), jnp.float32),
            pltpu.VMEM((2, RPB, D // 2), jnp.int32),
            pltpu.SemaphoreType.DMA,
            pltpu.SemaphoreType.DMA,
            pltpu.SemaphoreType.DMA,
            pltpu.SemaphoreType.DMA,
        ],
    )(emb_table)
    return pl.kernel(
        _bag_body,
        out_type=jax.ShapeDtypeStruct((B, D), jnp.float32),
        mesh=mesh,
        compiler_params=_SC_PARAMS,
        scratch_types=[
            pltpu.VMEM((2, BLK * NCH, IDX_CHUNK), jnp.int32),
            pltpu.VMEM((BLK, D), jnp.float32),
        ] + [pltpu.VMEM((IDX_CHUNK, D // 2), jnp.int32)] * (NSLOT * NCH)
          + [pltpu.SemaphoreType.DMA] * NSLOT,
    )(idx3, tabp)


TB = 1024


def _head_body(x_ref, pwt_ref, pb_ref, w1_ref, b1_ref, w2_ref, b2_ref,
               w3_ref, b3_ref, pol_ref, val_ref):
    x = x_ref[...]
    hi = lax.Precision.HIGHEST
    pol_ref[...] = (
        lax.dot_general(x, pwt_ref[...], (((1,), (0,)), ((), ())),
                        precision=lax.Precision.DEFAULT)
        + pb_ref[...]
    )
    h = jnp.clip(
        lax.dot_general(x, w1_ref[...], (((1,), (0,)), ((), ())), precision=hi)
        + b1_ref[...], 0.0, 1.0)
    h = jnp.clip(
        lax.dot_general(h, w2_ref[...], (((1,), (0,)), ((), ())), precision=hi)
        + b2_ref[...], 0.0, 1.0)
    val_ref[...] = jnp.tanh(
        lax.dot_general(h, w3_ref[...], (((1,), (0,)), ((), ())), precision=hi)
        + b3_ref[...])


def _heads(x, pw, pb, v1w, v1b, v2w, v2b, v3w, v3b):
    np_ = pw.shape[0]
    full = lambda shape: pl.BlockSpec(shape, lambda i: (0, 0))
    return pl.pallas_call(
        _head_body,
        grid=(B // TB,),
        in_specs=[
            pl.BlockSpec((TB, D), lambda i: (i, 0)),
            full((D, np_)),
            full((1, np_)),
            full((D, 32)),
            full((1, 32)),
            full((32, 32)),
            full((1, 32)),
            full((32, 1)),
            full((1, 1)),
        ],
        out_specs=[
            pl.BlockSpec((TB, np_), lambda i: (i, 0)),
            pl.BlockSpec((TB, 1), lambda i: (i, 0)),
        ],
        out_shape=[
            jax.ShapeDtypeStruct((B, np_), jnp.float32),
            jax.ShapeDtypeStruct((B, 1), jnp.float32),
        ],
    )(
        x, pw.T, pb.reshape(1, np_), v1w.T, v1b.reshape(1, 32),
        v2w.T, v2b.reshape(1, 32), v3w.T, v3b.reshape(1, 1),
    )


def kernel(indices, emb_table, pw, pb, v1w, v1b, v2w, v2b, v3w, v3b):
    x = _embedding_bag(indices, emb_table)
    pol, val = _heads(x, pw, pb, v1w, v1b, v2w, v2b, v3w, v3b)
    return (pol, val)


# speedup vs baseline: 1.7719x; 1.0006x over previous
import functools

import numpy as np
import jax
import jax.numpy as jnp
from jax import lax
from jax.experimental import pallas as pl
from jax.experimental.pallas import tpu as pltpu
from jax.experimental.pallas import tpu_sc as plsc

NC, NS, LANES = 2, 16, 16
NW = NC * NS

B, L, D = 16384, 200, 128
F = 2 * 225 * 225
SPW = B // NW
IDX_CHUNK = 100
NCH = L // IDX_CHUNK
BLK = 32
NBLK = SPW // BLK
NSLOT = 8

RPB = 128
NFULL = F // RPB
REM = F - NFULL * RPB
NITER_A = (NFULL + NW - 1) // NW

_SC_PARAMS = pltpu.CompilerParams(
    needs_layout_passes=False, use_tc_tiling_on_sc=False)


def _pack_body(tab_hbm, out_hbm, in_v, out_v, sem_i0, sem_i1, sem_o0, sem_o1):
    cid = lax.axis_index("c")
    sid = lax.axis_index("s")
    wid = sid * NC + cid
    sem_i = (sem_i0, sem_i1)
    sem_o = (sem_o0, sem_o1)
    hi_mask = jnp.full((LANES,), -65536, jnp.int32)
    s16 = jnp.full((LANES,), 16, jnp.int32)
    one = jnp.full((LANES,), 1, jnp.int32)
    half = jnp.full((LANES,), 0x7FFF, jnp.int32)

    def rne(bits):
        return bits + half + jnp.bitwise_and(
            lax.shift_right_logical(bits, s16), one)

    def pack_rows(slot, nrows):
        def r_body(r, carry):
            for k in range(D // 32):
                a = in_v[slot, r, pl.ds(k * 32, LANES)]
                bb = in_v[slot, r, pl.ds(k * 32 + 16, LANES)]
                ra = rne(plsc.bitcast(a, jnp.int32))
                rb = rne(plsc.bitcast(bb, jnp.int32))
                out_v[slot, r, pl.ds(k * LANES, LANES)] = jnp.bitwise_or(
                    lax.shift_right_logical(ra, s16),
                    jnp.bitwise_and(rb, hi_mask))
            return carry

        lax.fori_loop(0, nrows, r_body, 0)

    def fire_in(t, slot):
        pltpu.async_copy(tab_hbm.at[pl.ds(t * RPB, RPB)],
                         in_v.at[slot], sem_i[slot])

    def c_body(i, slot):
        t = wid + i * NW

        @pl.when(t + NW < NFULL)
        def _():
            fire_in(t + NW, 1 - slot)

        @pl.when(t < NFULL)
        def _():
            pltpu.make_async_copy(tab_hbm.at[pl.ds(0, RPB)],
                                  in_v.at[slot], sem_i[slot]).wait()

            @pl.when(t - 2 * NW >= 0)
            def _():
                pltpu.make_async_copy(out_v.at[slot],
                                      out_hbm.at[pl.ds(0, RPB)],
                                      sem_o[slot]).wait()

            pack_rows(slot, RPB)
            pltpu.async_copy(out_v.at[slot],
                             out_hbm.at[pl.ds(t * RPB, RPB)], sem_o[slot])

    fire_in(wid, 0)
    def c2_body(i2, carry):
        c_body(2 * i2, 0)
        c_body(2 * i2 + 1, 1)
        return carry

    assert NITER_A % 2 == 1
    lax.fori_loop(0, NITER_A // 2, c2_body, 0)
    c_body(NITER_A - 1, (NITER_A - 1) % 2)

    for j in (NITER_A - 2, NITER_A - 1):
        t = wid + j * NW

        @pl.when(t < NFULL)
        def _():
            pltpu.make_async_copy(out_v.at[j % 2],
                                  out_hbm.at[pl.ds(0, RPB)],
                                  sem_o[j % 2]).wait()

    @pl.when(wid == 0)
    def _():
        pltpu.sync_copy(tab_hbm.at[pl.ds(NFULL * RPB, REM)],
                        in_v.at[0].at[pl.ds(0, REM)])
        pack_rows(0, REM)
        pltpu.sync_copy(out_v.at[0].at[pl.ds(0, REM)],
                        out_hbm.at[pl.ds(NFULL * RPB, REM)])


def _bag_body(idx_hbm, tab_hbm, x_hbm, idx_v, acc_v, *bufs_and_sems):
    bufs = bufs_and_sems[: NSLOT * NCH]
    sems = bufs_and_sems[NSLOT * NCH:]
    cid = lax.axis_index("c")
    sid = lax.axis_index("s")
    wid = sid * NC + cid

    def fire(t, slot):
        bank = lax.rem(lax.div(t, BLK), 2)
        row0 = lax.rem(t, BLK) * NCH
        for c in range(NCH):
            pltpu.async_copy(tab_hbm.at[idx_v.at[bank].at[row0 + c]],
                             bufs[slot * NCH + c], sems[slot])

    def drain(slot):
        for c in range(NCH):
            pltpu.make_async_copy(tab_hbm.at[idx_v.at[0].at[0]],
                                  bufs[slot * NCH + c], sems[slot]).wait()

    def stage_idx(nb):
        bank = lax.rem(nb, 2)
        pltpu.sync_copy(idx_hbm.at[wid * NBLK + nb], idx_v.at[bank])

    hi_mask = jnp.full((LANES,), -65536, jnp.int32)
    shift16 = jnp.full((LANES,), 16, jnp.int32)

    def accum(s, slot):
        def r_body(r2, accs):
            out = list(accs)
            for rr in range(2):
                for c in range(NCH):
                    for k in range(D // 32):
                        v = bufs[slot * NCH + c][2 * r2 + rr,
                                                 pl.ds(k * LANES, LANES)]
                        ev = plsc.bitcast(lax.shift_left(v, shift16), jnp.float32)
                        od = plsc.bitcast(jnp.bitwise_and(v, hi_mask), jnp.float32)
                        out[2 * k] = out[2 * k] + ev
                        out[2 * k + 1] = out[2 * k + 1] + od
            return tuple(out)

        init = tuple(jnp.zeros((LANES,), jnp.float32) for _ in range(D // LANES))
        accs = lax.fori_loop(0, IDX_CHUNK // 2, r_body, init)
        arow = lax.rem(s, BLK)
        for k in range(D // LANES):
            acc_v[arow, pl.ds(k * LANES, LANES)] = accs[k]

    stage_idx(0)
    for b in range(NSLOT):
        fire(b, b)

    def g_body(g2, carry):
        g = g2 * NSLOT
        for b in range(NSLOT):
            s = g + b
            t = s + NSLOT
            drain(b)
            accum(s, b)

            @pl.when(jnp.logical_and(lax.rem(t, BLK) == 0, t < SPW))
            def _():
                stage_idx(t // BLK)

            @pl.when(t < SPW)
            def _():
                fire(t, b)

        s_last = g + NSLOT - 1

        @pl.when(lax.rem(s_last, BLK) == BLK - 1)
        def _():
            base = pl.multiple_of(wid * SPW + s_last - (BLK - 1), BLK)
            pltpu.sync_copy(acc_v, x_hbm.at[pl.ds(base, BLK)])

        return carry

    lax.fori_loop(0, SPW // NSLOT, g_body, 0)


def _embedding_bag(indices, emb_table):
    idx3 = indices.astype(jnp.int32).reshape(B // BLK, BLK * NCH, IDX_CHUNK)
    mesh = plsc.VectorSubcoreMesh(
        core_axis_name="c", subcore_axis_name="s", num_cores=NC, num_subcores=NS
    )
    tabp = pl.kernel(
        _pack_body,
        out_type=jax.ShapeDtypeStruct((F, D // 2), jnp.int32),
        mesh=mesh,
        compiler_params=_SC_PARAMS,
        scratch_types=[
            pltpu.VMEM((2, RPB, D), jnp.float32),
            pltpu.VMEM((2, RPB, D // 2), jnp.int32),
            pltpu.SemaphoreType.DMA,
            pltpu.SemaphoreType.DMA,
            pltpu.SemaphoreType.DMA,
            pltpu.SemaphoreType.DMA,
        ],
    )(emb_table)
    return pl.kernel(
        _bag_body,
        out_type=jax.ShapeDtypeStruct((B, D), jnp.float32),
        mesh=mesh,
        compiler_params=_SC_PARAMS,
        scratch_types=[
            pltpu.VMEM((2, BLK * NCH, IDX_CHUNK), jnp.int32),
            pltpu.VMEM((BLK, D), jnp.float32),
        ] + [pltpu.VMEM((IDX_CHUNK, D // 2), jnp.int32)] * (NSLOT * NCH)
          + [pltpu.SemaphoreType.DMA] * NSLOT,
    )(idx3, tabp)


TB = 1024


def _head_body(x_ref, pwt_ref, pb_ref, w1_ref, b1_ref, w2_ref, b2_ref,
               w3_ref, b3_ref, pol_ref, val_ref):
    x = x_ref[...]
    hi = lax.Precision.HIGHEST
    pol_ref[...] = (
        lax.dot_general(x, pwt_ref[...], (((1,), (0,)), ((), ())),
                        precision=lax.Precision.DEFAULT)
        + pb_ref[...]
    )
    h = jnp.clip(
        lax.dot_general(x, w1_ref[...], (((1,), (0,)), ((), ())), precision=hi)
        + b1_ref[...], 0.0, 1.0)
    h = jnp.clip(
        lax.dot_general(h, w2_ref[...], (((1,), (0,)), ((), ())), precision=hi)
        + b2_ref[...], 0.0, 1.0)
    val_ref[...] = jnp.tanh(
        lax.dot_general(h, w3_ref[...], (((1,), (0,)), ((), ())), precision=hi)
        + b3_ref[...])


def _heads(x, pw, pb, v1w, v1b, v2w, v2b, v3w, v3b):
    np_ = pw.shape[0]
    full = lambda shape: pl.BlockSpec(shape, lambda i: (0, 0))
    return pl.pallas_call(
        _head_body,
        grid=(B // TB,),
        in_specs=[
            pl.BlockSpec((TB, D), lambda i: (i, 0)),
            full((D, np_)),
            full((1, np_)),
            full((D, 32)),
            full((1, 32)),
            full((32, 32)),
            full((1, 32)),
            full((32, 1)),
            full((1, 1)),
        ],
        out_specs=[
            pl.BlockSpec((TB, np_), lambda i: (i, 0)),
            pl.BlockSpec((TB, 1), lambda i: (i, 0)),
        ],
        out_shape=[
            jax.ShapeDtypeStruct((B, np_), jnp.float32),
            jax.ShapeDtypeStruct((B, 1), jnp.float32),
        ],
    )(
        x, pw.T, pb.reshape(1, np_), v1w.T, v1b.reshape(1, 32),
        v2w.T, v2b.reshape(1, 32), v3w.T, v3b.reshape(1, 1),
    )


def kernel(indices, emb_table, pw, pb, v1w, v1b, v2w, v2b, v3w, v3b):
    x = _embedding_bag(indices, emb_table)
    pol, val = _heads(x, pw, pb, v1w, v1b, v2w, v2b, v3w, v3b)
    return (pol, val)


# speedup vs baseline: 2.0175x; 1.1386x over previous
import functools

import numpy as np
import jax
import jax.numpy as jnp
from jax import lax
from jax.experimental import pallas as pl
from jax.experimental.pallas import tpu as pltpu
from jax.experimental.pallas import tpu_sc as plsc

NC, NS, LANES = 2, 16, 16
NW = NC * NS

B, L, D = 16384, 200, 128
F = 2 * 225 * 225
SPW = B // NW
IDX_CHUNK = 100
NCH = L // IDX_CHUNK
BLK = 32
NBLK = SPW // BLK
NSLOT = 8

RPB = 128
NFULL = F // RPB
REM = F - NFULL * RPB
NITER_A = (NFULL + NW - 1) // NW

_SC_PARAMS = pltpu.CompilerParams(
    needs_layout_passes=False, use_tc_tiling_on_sc=False)


def _pack_body(tab_hbm, out_hbm, in_v, out_v, sem_i0, sem_i1, sem_o0, sem_o1):
    cid = lax.axis_index("c")
    sid = lax.axis_index("s")
    wid = sid * NC + cid
    sem_i = (sem_i0, sem_i1)
    sem_o = (sem_o0, sem_o1)
    hi_mask = jnp.full((LANES,), -65536, jnp.int32)
    s16 = jnp.full((LANES,), 16, jnp.int32)
    one = jnp.full((LANES,), 1, jnp.int32)
    half = jnp.full((LANES,), 0x7FFF, jnp.int32)

    def rne(bits):
        return bits + half + jnp.bitwise_and(
            lax.shift_right_logical(bits, s16), one)

    def pack_rows(slot, nrows):
        def r_body(r, carry):
            for k in range(D // 32):
                a = in_v[slot, r, pl.ds(k * 32, LANES)]
                bb = in_v[slot, r, pl.ds(k * 32 + 16, LANES)]
                ra = rne(plsc.bitcast(a, jnp.int32))
                rb = rne(plsc.bitcast(bb, jnp.int32))
                out_v[slot, r, pl.ds(k * LANES, LANES)] = jnp.bitwise_or(
                    lax.shift_right_logical(ra, s16),
                    jnp.bitwise_and(rb, hi_mask))
            return carry

        lax.fori_loop(0, nrows, r_body, 0)

    def fire_in(t, slot):
        pltpu.async_copy(tab_hbm.at[pl.ds(t * RPB, RPB)],
                         in_v.at[slot], sem_i[slot])

    def c_body(i, slot):
        t = wid + i * NW

        @pl.when(t + NW < NFULL)
        def _():
            fire_in(t + NW, 1 - slot)

        @pl.when(t < NFULL)
        def _():
            pltpu.make_async_copy(tab_hbm.at[pl.ds(0, RPB)],
                                  in_v.at[slot], sem_i[slot]).wait()

            @pl.when(t - 2 * NW >= 0)
            def _():
                pltpu.make_async_copy(out_v.at[slot],
                                      out_hbm.at[pl.ds(0, RPB)],
                                      sem_o[slot]).wait()

            pack_rows(slot, RPB)
            pltpu.async_copy(out_v.at[slot],
                             out_hbm.at[pl.ds(t * RPB, RPB)], sem_o[slot])

    fire_in(wid, 0)
    def c2_body(i2, carry):
        c_body(2 * i2, 0)
        c_body(2 * i2 + 1, 1)
        return carry

    assert NITER_A % 2 == 1
    lax.fori_loop(0, NITER_A // 2, c2_body, 0)
    c_body(NITER_A - 1, (NITER_A - 1) % 2)

    for j in (NITER_A - 2, NITER_A - 1):
        t = wid + j * NW

        @pl.when(t < NFULL)
        def _():
            pltpu.make_async_copy(out_v.at[j % 2],
                                  out_hbm.at[pl.ds(0, RPB)],
                                  sem_o[j % 2]).wait()

    @pl.when(wid == 0)
    def _():
        pltpu.sync_copy(tab_hbm.at[pl.ds(NFULL * RPB, REM)],
                        in_v.at[0].at[pl.ds(0, REM)])
        pack_rows(0, REM)
        pltpu.sync_copy(out_v.at[0].at[pl.ds(0, REM)],
                        out_hbm.at[pl.ds(NFULL * RPB, REM)])


def _bag_body(idx_hbm, tab_hbm, x_hbm, idx_v, acc_v, *bufs_and_sems):
    bufs = bufs_and_sems[: NSLOT * NCH]
    sems = bufs_and_sems[NSLOT * NCH:]
    cid = lax.axis_index("c")
    sid = lax.axis_index("s")
    wid = sid * NC + cid

    def fire(t, slot):
        bank = lax.rem(lax.div(t, BLK), 2)
        row0 = lax.rem(t, BLK) * NCH
        for c in range(NCH):
            pltpu.async_copy(tab_hbm.at[idx_v.at[bank].at[row0 + c]],
                             bufs[slot * NCH + c], sems[slot])

    def drain(slot):
        for c in range(NCH):
            pltpu.make_async_copy(tab_hbm.at[idx_v.at[0].at[0]],
                                  bufs[slot * NCH + c], sems[slot]).wait()

    def stage_idx(nb):
        bank = lax.rem(nb, 2)
        pltpu.sync_copy(idx_hbm.at[wid * NBLK + nb], idx_v.at[bank])

    hi_mask = jnp.full((LANES,), -65536, jnp.int32)
    shift16 = jnp.full((LANES,), 16, jnp.int32)

    def accum(s, slot):
        def r_body(r2, accs):
            out = list(accs)
            for rr in range(2):
                for c in range(NCH):
                    for k in range(D // 32):
                        v = bufs[slot * NCH + c][2 * r2 + rr,
                                                 pl.ds(k * LANES, LANES)]
                        ev = plsc.bitcast(lax.shift_left(v, shift16), jnp.float32)
                        od = plsc.bitcast(v, jnp.float32)
                        out[2 * k] = out[2 * k] + ev
                        out[2 * k + 1] = out[2 * k + 1] + od
            return tuple(out)

        init = tuple(jnp.zeros((LANES,), jnp.float32) for _ in range(D // LANES))
        accs = lax.fori_loop(0, IDX_CHUNK // 2, r_body, init)
        arow = lax.rem(s, BLK)
        for k in range(D // LANES):
            acc_v[arow, pl.ds(k * LANES, LANES)] = accs[k]

    stage_idx(0)
    for b in range(NSLOT):
        fire(b, b)

    def g_body(g2, carry):
        g = g2 * NSLOT
        for b in range(NSLOT):
            s = g + b
            t = s + NSLOT
            drain(b)
            accum(s, b)

            @pl.when(jnp.logical_and(lax.rem(t, BLK) == 0, t < SPW))
            def _():
                stage_idx(t // BLK)

            @pl.when(t < SPW)
            def _():
                fire(t, b)

        s_last = g + NSLOT - 1

        @pl.when(lax.rem(s_last, BLK) == BLK - 1)
        def _():
            base = pl.multiple_of(wid * SPW + s_last - (BLK - 1), BLK)
            pltpu.sync_copy(acc_v, x_hbm.at[pl.ds(base, BLK)])

        return carry

    lax.fori_loop(0, SPW // NSLOT, g_body, 0)


def _embedding_bag(indices, emb_table):
    idx3 = indices.astype(jnp.int32).reshape(B // BLK, BLK * NCH, IDX_CHUNK)
    mesh = plsc.VectorSubcoreMesh(
        core_axis_name="c", subcore_axis_name="s", num_cores=NC, num_subcores=NS
    )
    tabp = pl.kernel(
        _pack_body,
        out_type=jax.ShapeDtypeStruct((F, D // 2), jnp.int32),
        mesh=mesh,
        compiler_params=_SC_PARAMS,
        scratch_types=[
            pltpu.VMEM((2, RPB, D), jnp.float32),
            pltpu.VMEM((2, RPB, D // 2), jnp.int32),
            pltpu.SemaphoreType.DMA,
            pltpu.SemaphoreType.DMA,
            pltpu.SemaphoreType.DMA,
            pltpu.SemaphoreType.DMA,
        ],
    )(emb_table)
    return pl.kernel(
        _bag_body,
        out_type=jax.ShapeDtypeStruct((B, D), jnp.float32),
        mesh=mesh,
        compiler_params=_SC_PARAMS,
        scratch_types=[
            pltpu.VMEM((2, BLK * NCH, IDX_CHUNK), jnp.int32),
            pltpu.VMEM((BLK, D), jnp.float32),
        ] + [pltpu.VMEM((IDX_CHUNK, D // 2), jnp.int32)] * (NSLOT * NCH)
          + [pltpu.SemaphoreType.DMA] * NSLOT,
    )(idx3, tabp)


TB = 1024


def _head_body(x_ref, pwt_ref, pb_ref, w1_ref, b1_ref, w2_ref, b2_ref,
               w3_ref, b3_ref, pol_ref, val_ref):
    x = x_ref[...]
    hi = lax.Precision.HIGHEST
    pol_ref[...] = (
        lax.dot_general(x, pwt_ref[...], (((1,), (0,)), ((), ())),
                        precision=lax.Precision.DEFAULT)
        + pb_ref[...]
    )
    h = jnp.clip(
        lax.dot_general(x, w1_ref[...], (((1,), (0,)), ((), ())), precision=hi)
        + b1_ref[...], 0.0, 1.0)
    h = jnp.clip(
        lax.dot_general(h, w2_ref[...], (((1,), (0,)), ((), ())), precision=hi)
        + b2_ref[...], 0.0, 1.0)
    val_ref[...] = jnp.tanh(
        lax.dot_general(h, w3_ref[...], (((1,), (0,)), ((), ())), precision=hi)
        + b3_ref[...])


def _heads(x, pw, pb, v1w, v1b, v2w, v2b, v3w, v3b):
    np_ = pw.shape[0]
    full = lambda shape: pl.BlockSpec(shape, lambda i: (0, 0))
    return pl.pallas_call(
        _head_body,
        grid=(B // TB,),
        in_specs=[
            pl.BlockSpec((TB, D), lambda i: (i, 0)),
            full((D, np_)),
            full((1, np_)),
            full((D, 32)),
            full((1, 32)),
            full((32, 32)),
            full((1, 32)),
            full((32, 1)),
            full((1, 1)),
        ],
        out_specs=[
            pl.BlockSpec((TB, np_), lambda i: (i, 0)),
            pl.BlockSpec((TB, 1), lambda i: (i, 0)),
        ],
        out_shape=[
            jax.ShapeDtypeStruct((B, np_), jnp.float32),
            jax.ShapeDtypeStruct((B, 1), jnp.float32),
        ],
    )(
        x, pw.T, pb.reshape(1, np_), v1w.T, v1b.reshape(1, 32),
        v2w.T, v2b.reshape(1, 32), v3w.T, v3b.reshape(1, 1),
    )


def kernel(indices, emb_table, pw, pb, v1w, v1b, v2w, v2b, v3w, v3b):
    x = _embedding_bag(indices, emb_table)
    pol, val = _heads(x, pw, pb, v1w, v1b, v2w, v2b, v3w, v3b)
    return (pol, val)


# speedup vs baseline: 2.1393x; 1.0604x over previous
import functools

import numpy as np
import jax
import jax.numpy as jnp
from jax import lax
from jax.experimental import pallas as pl
from jax.experimental.pallas import tpu as pltpu
from jax.experimental.pallas import tpu_sc as plsc

NC, NS, LANES = 2, 16, 16
NW = NC * NS

B, L, D = 16384, 200, 128
F = 2 * 225 * 225
SPW = B // NW
IDX_CHUNK = 100
NCH = L // IDX_CHUNK
BLK = 32
NBLK = SPW // BLK
NSLOT = 8

RPB = 256
NFULL = F // RPB
REM = F - NFULL * RPB
NITER_A = (NFULL + NW - 1) // NW

_SC_PARAMS = pltpu.CompilerParams(
    needs_layout_passes=False, use_tc_tiling_on_sc=False)


def _pack_body(tab_hbm, out_hbm, in_v, out_v, sem_i0, sem_i1, sem_o0, sem_o1):
    cid = lax.axis_index("c")
    sid = lax.axis_index("s")
    wid = sid * NC + cid
    sem_i = (sem_i0, sem_i1)
    sem_o = (sem_o0, sem_o1)
    hi_mask = jnp.full((LANES,), -65536, jnp.int32)
    s16 = jnp.full((LANES,), 16, jnp.int32)

    def pack_rows(slot, nrows):
        def r_body(r, carry):
            for k in range(D // 32):
                a = in_v[slot, r, pl.ds(k * 32, LANES)]
                bb = in_v[slot, r, pl.ds(k * 32 + 16, LANES)]
                out_v[slot, r, pl.ds(k * LANES, LANES)] = jnp.bitwise_or(
                    lax.shift_right_logical(plsc.bitcast(a, jnp.int32), s16),
                    jnp.bitwise_and(plsc.bitcast(bb, jnp.int32), hi_mask))
            return carry

        lax.fori_loop(0, nrows, r_body, 0)

    def fire_in(t, slot):
        pltpu.async_copy(tab_hbm.at[pl.ds(t * RPB, RPB)],
                         in_v.at[slot], sem_i[slot])

    def c_body(i, slot):
        t = wid + i * NW

        @pl.when(t + NW < NFULL)
        def _():
            fire_in(t + NW, 1 - slot)

        @pl.when(t < NFULL)
        def _():
            pltpu.make_async_copy(tab_hbm.at[pl.ds(0, RPB)],
                                  in_v.at[slot], sem_i[slot]).wait()

            @pl.when(t - 2 * NW >= 0)
            def _():
                pltpu.make_async_copy(out_v.at[slot],
                                      out_hbm.at[pl.ds(0, RPB)],
                                      sem_o[slot]).wait()

            pack_rows(slot, RPB)
            pltpu.async_copy(out_v.at[slot],
                             out_hbm.at[pl.ds(t * RPB, RPB)], sem_o[slot])

    fire_in(wid, 0)
    def c2_body(i2, carry):
        c_body(2 * i2, 0)
        c_body(2 * i2 + 1, 1)
        return carry

    assert NITER_A % 2 == 1
    lax.fori_loop(0, NITER_A // 2, c2_body, 0)
    c_body(NITER_A - 1, (NITER_A - 1) % 2)

    for j in (NITER_A - 2, NITER_A - 1):
        t = wid + j * NW

        @pl.when(t < NFULL)
        def _():
            pltpu.make_async_copy(out_v.at[j % 2],
                                  out_hbm.at[pl.ds(0, RPB)],
                                  sem_o[j % 2]).wait()

    @pl.when(wid == 0)
    def _():
        pltpu.sync_copy(tab_hbm.at[pl.ds(NFULL * RPB, REM)],
                        in_v.at[0].at[pl.ds(0, REM)])
        pack_rows(0, REM)
        pltpu.sync_copy(out_v.at[0].at[pl.ds(0, REM)],
                        out_hbm.at[pl.ds(NFULL * RPB, REM)])


def _bag_body(idx_hbm, tab_hbm, x_hbm, idx_v, acc_v, *bufs_and_sems):
    bufs = bufs_and_sems[: NSLOT * NCH]
    sems = bufs_and_sems[NSLOT * NCH:]
    cid = lax.axis_index("c")
    sid = lax.axis_index("s")
    wid = sid * NC + cid

    def fire(t, slot):
        bank = lax.rem(lax.div(t, BLK), 2)
        row0 = lax.rem(t, BLK) * NCH
        for c in range(NCH):
            pltpu.async_copy(tab_hbm.at[idx_v.at[bank].at[row0 + c]],
                             bufs[slot * NCH + c], sems[slot])

    def drain(slot):
        for c in range(NCH):
            pltpu.make_async_copy(tab_hbm.at[idx_v.at[0].at[0]],
                                  bufs[slot * NCH + c], sems[slot]).wait()

    def stage_idx(nb):
        bank = lax.rem(nb, 2)
        pltpu.sync_copy(idx_hbm.at[wid * NBLK + nb], idx_v.at[bank])

    hi_mask = jnp.full((LANES,), -65536, jnp.int32)
    shift16 = jnp.full((LANES,), 16, jnp.int32)

    def accum(s, slot):
        def r_body(r2, accs):
            out = list(accs)
            for rr in range(2):
                for c in range(NCH):
                    for k in range(D // 32):
                        v = bufs[slot * NCH + c][2 * r2 + rr,
                                                 pl.ds(k * LANES, LANES)]
                        ev = plsc.bitcast(lax.shift_left(v, shift16), jnp.float32)
                        od = plsc.bitcast(v, jnp.float32)
                        out[2 * k] = out[2 * k] + ev
                        out[2 * k + 1] = out[2 * k + 1] + od
            return tuple(out)

        init = tuple(jnp.zeros((LANES,), jnp.float32) for _ in range(D // LANES))
        accs = lax.fori_loop(0, IDX_CHUNK // 2, r_body, init)
        abase = lax.rem(s, BLK) * D
        for k in range(D // LANES):
            acc_v[pl.ds(abase + k * LANES, LANES)] = accs[k]

    stage_idx(0)
    for b in range(NSLOT):
        fire(b, b)

    def g_body(g2, carry):
        g = g2 * NSLOT
        for b in range(NSLOT):
            s = g + b
            t = s + NSLOT
            drain(b)
            accum(s, b)

            @pl.when(jnp.logical_and(lax.rem(t, BLK) == 0, t < SPW))
            def _():
                stage_idx(t // BLK)

            @pl.when(t < SPW)
            def _():
                fire(t, b)

        s_last = g + NSLOT - 1

        @pl.when(lax.rem(s_last, BLK) == BLK - 1)
        def _():
            base = pl.multiple_of((wid * SPW + s_last - (BLK - 1)) * D, BLK * D)
            pltpu.sync_copy(acc_v, x_hbm.at[pl.ds(base, BLK * D)])

        return carry

    lax.fori_loop(0, SPW // NSLOT, g_body, 0)


def _embedding_bag(indices, emb_table):
    idx3 = indices.astype(jnp.int32).reshape(B // BLK, BLK * NCH, IDX_CHUNK)
    mesh = plsc.VectorSubcoreMesh(
        core_axis_name="c", subcore_axis_name="s", num_cores=NC, num_subcores=NS
    )
    tabp = pl.kernel(
        _pack_body,
        out_type=jax.ShapeDtypeStruct((F, D // 2), jnp.int32),
        mesh=mesh,
        compiler_params=_SC_PARAMS,
        scratch_types=[
            pltpu.VMEM((2, RPB, D), jnp.float32),
            pltpu.VMEM((2, RPB, D // 2), jnp.int32),
            pltpu.SemaphoreType.DMA,
            pltpu.SemaphoreType.DMA,
            pltpu.SemaphoreType.DMA,
            pltpu.SemaphoreType.DMA,
        ],
    )(emb_table)
    x1d = pl.kernel(
        _bag_body,
        out_type=jax.ShapeDtypeStruct((B * D,), jnp.float32),
        mesh=mesh,
        compiler_params=_SC_PARAMS,
        scratch_types=[
            pltpu.VMEM((2, BLK * NCH, IDX_CHUNK), jnp.int32),
            pltpu.VMEM((BLK * D,), jnp.float32),
        ] + [pltpu.VMEM((IDX_CHUNK, D // 2), jnp.int32)] * (NSLOT * NCH)
          + [pltpu.SemaphoreType.DMA] * NSLOT,
    )(idx3, tabp)
    return x1d.reshape(B, D)


TB = 1024


def _head_body(x_ref, pwt_ref, pb_ref, w1_ref, b1_ref, w2_ref, b2_ref,
               w3_ref, b3_ref, pol_ref, val_ref):
    x = x_ref[...]
    hi = lax.Precision.HIGHEST
    pol_ref[...] = (
        lax.dot_general(x, pwt_ref[...], (((1,), (0,)), ((), ())),
                        precision=lax.Precision.DEFAULT)
        + pb_ref[...]
    )
    h = jnp.clip(
        lax.dot_general(x, w1_ref[...], (((1,), (0,)), ((), ())), precision=hi)
        + b1_ref[...], 0.0, 1.0)
    h = jnp.clip(
        lax.dot_general(h, w2_ref[...], (((1,), (0,)), ((), ())), precision=hi)
        + b2_ref[...], 0.0, 1.0)
    val_ref[...] = jnp.tanh(
        lax.dot_general(h, w3_ref[...], (((1,), (0,)), ((), ())), precision=hi)
        + b3_ref[...])


def _heads(x, pw, pb, v1w, v1b, v2w, v2b, v3w, v3b):
    np_ = pw.shape[0]
    full = lambda shape: pl.BlockSpec(shape, lambda i: (0, 0))
    return pl.pallas_call(
        _head_body,
        grid=(B // TB,),
        in_specs=[
            pl.BlockSpec((TB, D), lambda i: (i, 0)),
            full((D, np_)),
            full((1, np_)),
            full((D, 32)),
            full((1, 32)),
            full((32, 32)),
            full((1, 32)),
            full((32, 1)),
            full((1, 1)),
        ],
        out_specs=[
            pl.BlockSpec((TB, np_), lambda i: (i, 0)),
            pl.BlockSpec((TB, 1), lambda i: (i, 0)),
        ],
        out_shape=[
            jax.ShapeDtypeStruct((B, np_), jnp.float32),
            jax.ShapeDtypeStruct((B, 1), jnp.float32),
        ],
    )(
        x, pw.T, pb.reshape(1, np_), v1w.T, v1b.reshape(1, 32),
        v2w.T, v2b.reshape(1, 32), v3w.T, v3b.reshape(1, 1),
    )


def kernel(indices, emb_table, pw, pb, v1w, v1b, v2w, v2b, v3w, v3b):
    x = _embedding_bag(indices, emb_table)
    pol, val = _heads(x, pw, pb, v1w, v1b, v2w, v2b, v3w, v3b)
    return (pol, val)


# speedup vs baseline: 2.1401x; 1.0004x over previous
import functools

import numpy as np
import jax
import jax.numpy as jnp
from jax import lax
from jax.experimental import pallas as pl
from jax.experimental.pallas import tpu as pltpu
from jax.experimental.pallas import tpu_sc as plsc

NC, NS, LANES = 2, 16, 16
NW = NC * NS

B, L, D = 16384, 200, 128
F = 2 * 225 * 225
SPW = B // NW
IDX_CHUNK = 100
NCH = L // IDX_CHUNK
BLK = 32
NBLK = SPW // BLK
NSLOT = 8

RPB = 256
NFULL = F // RPB
REM = F - NFULL * RPB
NITER_A = (NFULL + NW - 1) // NW

_SC_PARAMS = pltpu.CompilerParams(
    needs_layout_passes=False, use_tc_tiling_on_sc=False)
_SC_PARAMS_PACK = pltpu.CompilerParams(needs_layout_passes=False)


def _pack_body(tab_hbm, out_hbm, in_v, out_v, sem_i0, sem_i1, sem_o0, sem_o1):
    cid = lax.axis_index("c")
    sid = lax.axis_index("s")
    wid = sid * NC + cid
    sem_i = (sem_i0, sem_i1)
    sem_o = (sem_o0, sem_o1)
    hi_mask = jnp.full((LANES,), -65536, jnp.int32)
    s16 = jnp.full((LANES,), 16, jnp.int32)

    def pack_rows(slot, nrows):
        def r_body(r, carry):
            for k in range(D // 32):
                a = in_v[slot, r, pl.ds(k * 32, LANES)]
                bb = in_v[slot, r, pl.ds(k * 32 + 16, LANES)]
                out_v[slot, pl.ds(r * (D // 2) + k * LANES, LANES)] = (
                    jnp.bitwise_or(
                        lax.shift_right_logical(plsc.bitcast(a, jnp.int32), s16),
                        jnp.bitwise_and(plsc.bitcast(bb, jnp.int32), hi_mask)))
            return carry

        lax.fori_loop(0, nrows, r_body, 0)

    def fire_in(t, slot):
        pltpu.async_copy(tab_hbm.at[pl.ds(t * RPB, RPB)],
                         in_v.at[slot], sem_i[slot])

    def c_body(i, slot):
        t = wid + i * NW

        @pl.when(t + NW < NFULL)
        def _():
            fire_in(t + NW, 1 - slot)

        @pl.when(t < NFULL)
        def _():
            pltpu.make_async_copy(tab_hbm.at[pl.ds(0, RPB)],
                                  in_v.at[slot], sem_i[slot]).wait()

            @pl.when(t - 2 * NW >= 0)
            def _():
                pltpu.make_async_copy(out_v.at[slot],
                                      out_hbm.at[pl.ds(0, RPB * (D // 2))],
                                      sem_o[slot]).wait()

            pack_rows(slot, RPB)
            pltpu.async_copy(
                out_v.at[slot],
                out_hbm.at[pl.ds(t * RPB * (D // 2), RPB * (D // 2))],
                sem_o[slot])

    fire_in(wid, 0)
    def c2_body(i2, carry):
        c_body(2 * i2, 0)
        c_body(2 * i2 + 1, 1)
        return carry

    assert NITER_A % 2 == 1
    lax.fori_loop(0, NITER_A // 2, c2_body, 0)
    c_body(NITER_A - 1, (NITER_A - 1) % 2)

    for j in (NITER_A - 2, NITER_A - 1):
        t = wid + j * NW

        @pl.when(t < NFULL)
        def _():
            pltpu.make_async_copy(out_v.at[j % 2],
                                  out_hbm.at[pl.ds(0, RPB * (D // 2))],
                                  sem_o[j % 2]).wait()

    @pl.when(wid == 0)
    def _():
        pltpu.sync_copy(tab_hbm.at[pl.ds(NFULL * RPB, REM)],
                        in_v.at[0].at[pl.ds(0, REM)])
        pack_rows(0, REM)
        pltpu.sync_copy(
            out_v.at[0].at[pl.ds(0, REM * (D // 2))],
            out_hbm.at[pl.ds(NFULL * RPB * (D // 2), REM * (D // 2))])


def _bag_body(idx_hbm, tab_hbm, x_hbm, idx_v, acc_v, *bufs_and_sems):
    bufs = bufs_and_sems[: NSLOT * NCH]
    sems = bufs_and_sems[NSLOT * NCH:]
    cid = lax.axis_index("c")
    sid = lax.axis_index("s")
    wid = sid * NC + cid

    def fire(t, slot):
        bank = lax.rem(lax.div(t, BLK), 2)
        row0 = lax.rem(t, BLK) * NCH
        for c in range(NCH):
            pltpu.async_copy(tab_hbm.at[idx_v.at[bank].at[row0 + c]],
                             bufs[slot * NCH + c], sems[slot])

    def drain(slot):
        for c in range(NCH):
            pltpu.make_async_copy(tab_hbm.at[idx_v.at[0].at[0]],
                                  bufs[slot * NCH + c], sems[slot]).wait()

    def stage_idx(nb):
        bank = lax.rem(nb, 2)
        pltpu.sync_copy(idx_hbm.at[wid * NBLK + nb], idx_v.at[bank])

    hi_mask = jnp.full((LANES,), -65536, jnp.int32)
    shift16 = jnp.full((LANES,), 16, jnp.int32)

    def accum(s, slot):
        def r_body(r2, accs):
            out = list(accs)
            for rr in range(2):
                for c in range(NCH):
                    for k in range(D // 32):
                        v = bufs[slot * NCH + c][2 * r2 + rr,
                                                 pl.ds(k * LANES, LANES)]
                        ev = plsc.bitcast(lax.shift_left(v, shift16), jnp.float32)
                        od = plsc.bitcast(v, jnp.float32)
                        out[2 * k] = out[2 * k] + ev
                        out[2 * k + 1] = out[2 * k + 1] + od
            return tuple(out)

        init = tuple(jnp.zeros((LANES,), jnp.float32) for _ in range(D // LANES))
        accs = lax.fori_loop(0, IDX_CHUNK // 2, r_body, init)
        abase = lax.rem(s, BLK) * D
        for k in range(D // LANES):
            acc_v[pl.ds(abase + k * LANES, LANES)] = accs[k]

    stage_idx(0)
    for b in range(NSLOT):
        fire(b, b)

    def g_body(g2, carry):
        g = g2 * NSLOT
        for b in range(NSLOT):
            s = g + b
            t = s + NSLOT
            drain(b)
            accum(s, b)

            @pl.when(jnp.logical_and(lax.rem(t, BLK) == 0, t < SPW))
            def _():
                stage_idx(t // BLK)

            @pl.when(t < SPW)
            def _():
                fire(t, b)

        s_last = g + NSLOT - 1

        @pl.when(lax.rem(s_last, BLK) == BLK - 1)
        def _():
            base = pl.multiple_of((wid * SPW + s_last - (BLK - 1)) * D, BLK * D)
            pltpu.sync_copy(acc_v, x_hbm.at[pl.ds(base, BLK * D)])

        return carry

    lax.fori_loop(0, SPW // NSLOT, g_body, 0)


def _embedding_bag(indices, emb_table):
    idx3 = indices.astype(jnp.int32).reshape(B // BLK, BLK * NCH, IDX_CHUNK)
    mesh = plsc.VectorSubcoreMesh(
        core_axis_name="c", subcore_axis_name="s", num_cores=NC, num_subcores=NS
    )
    tabp = pl.kernel(
        _pack_body,
        out_type=jax.ShapeDtypeStruct((F * (D // 2),), jnp.int32),
        mesh=mesh,
        compiler_params=_SC_PARAMS_PACK,
        scratch_types=[
            pltpu.VMEM((2, RPB, D), jnp.float32),
            pltpu.VMEM((2, RPB * (D // 2)), jnp.int32),
            pltpu.SemaphoreType.DMA,
            pltpu.SemaphoreType.DMA,
            pltpu.SemaphoreType.DMA,
            pltpu.SemaphoreType.DMA,
        ],
    )(emb_table).reshape(F, D // 2)
    x1d = pl.kernel(
        _bag_body,
        out_type=jax.ShapeDtypeStruct((B * D,), jnp.float32),
        mesh=mesh,
        compiler_params=_SC_PARAMS,
        scratch_types=[
            pltpu.VMEM((2, BLK * NCH, IDX_CHUNK), jnp.int32),
            pltpu.VMEM((BLK * D,), jnp.float32),
        ] + [pltpu.VMEM((IDX_CHUNK, D // 2), jnp.int32)] * (NSLOT * NCH)
          + [pltpu.SemaphoreType.DMA] * NSLOT,
    )(idx3, tabp)
    return x1d.reshape(B, D)


TB = 2048


def _head_body(x_ref, pwt_ref, pb_ref, w1_ref, b1_ref, w2_ref, b2_ref,
               w3_ref, b3_ref, pol_ref, val_ref):
    x = x_ref[...]
    hi = lax.Precision.HIGHEST
    pol_ref[...] = (
        lax.dot_general(x, pwt_ref[...], (((1,), (0,)), ((), ())),
                        precision=lax.Precision.DEFAULT)
        + pb_ref[...]
    )
    h = jnp.clip(
        lax.dot_general(x, w1_ref[...], (((1,), (0,)), ((), ())),
                        precision=lax.Precision.DEFAULT)
        + b1_ref[...], 0.0, 1.0)
    h = jnp.clip(
        lax.dot_general(h, w2_ref[...], (((1,), (0,)), ((), ())), precision=hi)
        + b2_ref[...], 0.0, 1.0)
    val_ref[...] = jnp.tanh(
        lax.dot_general(h, w3_ref[...], (((1,), (0,)), ((), ())), precision=hi)
        + b3_ref[...])


def _heads(x, pw, pb, v1w, v1b, v2w, v2b, v3w, v3b):
    np_ = pw.shape[0]
    full = lambda shape: pl.BlockSpec(shape, lambda i: (0, 0))
    return pl.pallas_call(
        _head_body,
        grid=(B // TB,),
        in_specs=[
            pl.BlockSpec((TB, D), lambda i: (i, 0)),
            full((D, np_)),
            full((1, np_)),
            full((D, 32)),
            full((1, 32)),
            full((32, 32)),
            full((1, 32)),
            full((32, 1)),
            full((1, 1)),
        ],
        out_specs=[
            pl.BlockSpec((TB, np_), lambda i: (i, 0)),
            pl.BlockSpec((TB, 1), lambda i: (i, 0)),
        ],
        out_shape=[
            jax.ShapeDtypeStruct((B, np_), jnp.float32),
            jax.ShapeDtypeStruct((B, 1), jnp.float32),
        ],
    )(
        x, pw.T, pb.reshape(1, np_), v1w.T, v1b.reshape(1, 32),
        v2w.T, v2b.reshape(1, 32), v3w.T, v3b.reshape(1, 1),
    )


def kernel(indices, emb_table, pw, pb, v1w, v1b, v2w, v2b, v3w, v3b):
    x = _embedding_bag(indices, emb_table)
    pol, val = _heads(x, pw, pb, v1w, v1b, v2w, v2b, v3w, v3b)
    return (pol, val)
